# baseline (speedup 1.0000x reference)
import functools

import jax
import jax.numpy as jnp
from jax import lax
from jax.experimental import pallas as pl
from jax.experimental.pallas import tpu as pltpu
from jax.experimental.pallas import tpu_sc as plsc

V, L, C = 8, 729, 1152
N = V * L
G = 4096
GT = G + 8
NT = 16
NCORE = 2
P = 368
PA = 384
NV = P // 16
W = 288
CHUNKS_PER_CORE = 2
LAST_BASE = N - P
DUP = NT * P - N


def _sc_body(feat, coord, out, grid_sh, hist_sh, recip_sh,
             featbuf, zerobuf, coordbuf, idx_sc, idx_ga, hist_v, recip_v,
             precip):
    core = lax.axis_index("c")
    sid = lax.axis_index("s")
    lanes = lax.iota(jnp.int32, 16)

    base = jnp.minimum(sid * P, LAST_BASE)

    zero16i = jnp.zeros((16,), jnp.int32)

    def _zero_hist(i, _):
        hist_v[i & 15, pl.ds((i >> 4) * 16, 16)] = zero16i
        return 0
    lax.fori_loop(0, 256, _zero_hist, 0)

    zero16f = jnp.zeros((16,), jnp.float32)
    for i in range(8):
        for c in range(W // 16):
            zerobuf[i, pl.ds(c * 16, 16)] = zero16f

    pltpu.sync_copy(coord.at[pl.ds(base * 3, P * 3)], coordbuf)

    ones_i = jnp.ones((16,), jnp.int32)
    sid_ok = sid < (NT - 1)
    for j in range(NV):
        cbase = j * 48
        fx = plsc.load_gather(coordbuf, [cbase + lanes * 3])
        fy = plsc.load_gather(coordbuf, [cbase + lanes * 3 + 1])
        fz = plsc.load_gather(coordbuf, [cbase + lanes * 3 + 2])
        iv = (fx.astype(jnp.int32) * 256 + fy.astype(jnp.int32) * 16
              + fz.astype(jnp.int32))
        valid = jnp.logical_or(sid_ok, (j * 16 + lanes) >= DUP)
        iv_sc = jnp.where(valid, iv, G + (lanes & 7))
        row, col = j // 8, (j % 8) * 16
        idx_sc[row, pl.ds(col, 16)] = iv_sc
        idx_ga[row, pl.ds(col, 16)] = iv
        plsc.addupdate_scatter(hist_v, [iv >> 8, iv & 255], ones_i,
                               mask=valid)
    idx_sc[2, pl.ds(112, 16)] = G + (lanes & 7)
    idx_ga[2, pl.ds(112, 16)] = G + (lanes & 7)

    pltpu.sync_copy(hist_v, hist_sh.at[sid])
    plsc.subcore_barrier()
    pltpu.sync_copy(hist_sh.at[:, sid], hist_v)
    for v in range(16):
        acc = hist_v[0, pl.ds(v * 16, 16)]
        for r in range(1, 16):
            acc = acc + hist_v[r, pl.ds(v * 16, 16)]
        cnt = jnp.maximum(acc.astype(jnp.float32), 1.0)
        precip[pl.ds(v * 16, 16)] = 1.0 / cnt
    pltpu.sync_copy(precip.at[pl.ds(0, 256)],
                    recip_sh.at[pl.ds(sid * 256, 256)])
    plsc.subcore_barrier()
    pltpu.sync_copy(recip_sh, recip_v)

    for j in range(NV):
        row, col = j // 8, (j % 8) * 16
        ivg = idx_ga[row, pl.ds(col, 16)]
        precip[pl.ds(j * 16, 16)] = plsc.load_gather(recip_v, [ivg])

    nb = [128, 128, P - 256]
    for k in range(CHUNKS_PER_CORE):
        col0 = core * (W * CHUNKS_PER_CORE) + k * W
        plsc.subcore_barrier()
        for i in range(32):
            pltpu.sync_copy(zerobuf, grid_sh.at[pl.ds(sid * 256 + i * 8,
                                                      8)])
        plsc.subcore_barrier()
        for r in range(PA // 128):
            pltpu.sync_copy(feat.at[pl.ds(base + r * 128, nb[r]),
                                    pl.ds(col0, W)],
                            featbuf.at[pl.ds(0, nb[r])])
            pltpu.sync_copy(featbuf, grid_sh.at[idx_sc.at[r]], add=True)
        plsc.subcore_barrier()
        for r in range(PA // 128):
            pltpu.sync_copy(grid_sh.at[idx_ga.at[r]], featbuf)

            def _scale(j, _, r=r):
                rv = precip[pl.ds(r * 128 + j, 16)][0]
                for c in range(W // 16):
                    featbuf[j, pl.ds(c * 16, 16)] = (
                        featbuf[j, pl.ds(c * 16, 16)] * rv)
                return 0
            lax.fori_loop(0, nb[r], _scale, 0)
            pltpu.sync_copy(featbuf.at[pl.ds(0, nb[r])],
                            out.at[pl.ds(base + r * 128, nb[r]),
                                   pl.ds(col0, W)])


@functools.partial(
    pl.kernel,
    out_type=jax.ShapeDtypeStruct((N, C), jnp.float32),
    mesh=plsc.VectorSubcoreMesh(core_axis_name="c", subcore_axis_name="s"),
    compiler_params=pltpu.CompilerParams(use_tc_tiling_on_sc=False,
                                         needs_layout_passes=False),
    scratch_types=[
        pltpu.VMEM_SHARED((GT, W), jnp.float32),
        pltpu.VMEM_SHARED((NT, NT, 256), jnp.int32),
        pltpu.VMEM_SHARED((G,), jnp.float32),
        pltpu.VMEM((128, W), jnp.float32),
        pltpu.VMEM((8, W), jnp.float32),
        pltpu.VMEM((P * 3,), jnp.float32),
        pltpu.VMEM((PA // 128, 128), jnp.int32),
        pltpu.VMEM((PA // 128, 128), jnp.int32),
        pltpu.VMEM((16, 256), jnp.int32),
        pltpu.VMEM((G,), jnp.float32),
        pltpu.VMEM((PA,), jnp.float32),
    ],
)
def _voxel_mean_sc(feat, coord, out, *scratch):
    _sc_body(feat, coord, out, *scratch)


def kernel(video_tensor, coord_info):
    feats = video_tensor.reshape(N, C)
    coords = coord_info.reshape(N * 3)
    out = _voxel_mean_sc(feats, coords)
    return out.reshape(V, L, C)


# speedup vs baseline: 1.1175x; 1.1175x over previous
import functools

import jax
import jax.numpy as jnp
from jax import lax
from jax.experimental import pallas as pl
from jax.experimental.pallas import tpu as pltpu
from jax.experimental.pallas import tpu_sc as plsc

V, L, C = 8, 729, 1152
N = V * L
G = 4096
GT = G + 8
NT = 16
NCORE = 2
P = 368
PA = 384
NV = P // 16
W = 128
NCHUNK = C // W
LAST_BASE = N - P
DUP = NT * P - N


def _sc_body(feat, coord, out, grid_sh, hist_sh, recip_sh,
             featbuf, zerobuf, coordbuf, idx_sc, idx_ga, hist_v, recip_v,
             precip):
    core = lax.axis_index("c")
    sid = lax.axis_index("s")
    lanes = lax.iota(jnp.int32, 16)

    base = jnp.minimum(sid * P, LAST_BASE)

    zero16i = jnp.zeros((16,), jnp.int32)

    def _zero_hist(i, _):
        hist_v[i & 15, pl.ds((i >> 4) * 16, 16)] = zero16i
        return 0
    lax.fori_loop(0, 256, _zero_hist, 0)

    zero16f = jnp.zeros((16,), jnp.float32)
    for i in range(16):
        for c in range(W // 16):
            zerobuf[i, pl.ds(c * 16, 16)] = zero16f

    pltpu.sync_copy(coord.at[pl.ds(base * 3, P * 3)], coordbuf)

    ones_i = jnp.ones((16,), jnp.int32)
    sid_ok = sid < (NT - 1)
    for j in range(NV):
        cbase = j * 48
        fx = plsc.load_gather(coordbuf, [cbase + lanes * 3])
        fy = plsc.load_gather(coordbuf, [cbase + lanes * 3 + 1])
        fz = plsc.load_gather(coordbuf, [cbase + lanes * 3 + 2])
        iv = (fx.astype(jnp.int32) * 256 + fy.astype(jnp.int32) * 16
              + fz.astype(jnp.int32))
        valid = jnp.logical_or(sid_ok, (j * 16 + lanes) >= DUP)
        iv_sc = jnp.where(valid, iv, G + (lanes & 7))
        row, col = j // 8, (j % 8) * 16
        idx_sc[row, pl.ds(col, 16)] = iv_sc
        idx_ga[row, pl.ds(col, 16)] = iv
        plsc.addupdate_scatter(hist_v, [iv >> 8, iv & 255], ones_i,
                               mask=valid)
    idx_sc[2, pl.ds(112, 16)] = G + (lanes & 7)
    idx_ga[2, pl.ds(112, 16)] = G + (lanes & 7)

    pltpu.sync_copy(hist_v, hist_sh.at[sid])
    plsc.subcore_barrier()
    pltpu.sync_copy(hist_sh.at[:, sid], hist_v)
    for v in range(16):
        acc = hist_v[0, pl.ds(v * 16, 16)]
        for r in range(1, 16):
            acc = acc + hist_v[r, pl.ds(v * 16, 16)]
        cnt = jnp.maximum(acc.astype(jnp.float32), 1.0)
        precip[pl.ds(v * 16, 16)] = 1.0 / cnt
    pltpu.sync_copy(precip.at[pl.ds(0, 256)],
                    recip_sh.at[pl.ds(sid * 256, 256)])
    plsc.subcore_barrier()
    pltpu.sync_copy(recip_sh, recip_v)

    for j in range(NV):
        row, col = j // 8, (j % 8) * 16
        ivg = idx_ga[row, pl.ds(col, 16)]
        precip[pl.ds(j * 16, 16)] = plsc.load_gather(recip_v, [ivg])

    nb = [128, 128, P - 256]
    for k in range((NCHUNK + 1) // 2):
        ch = k + core * 5
        active = ch < NCHUNK
        col0 = jnp.minimum(ch, NCHUNK - 1) * W
        plsc.subcore_barrier()

        @pl.when(active)
        def _zero():
            for i in range(16):
                pltpu.sync_copy(zerobuf,
                                grid_sh.at[pl.ds(sid * 256 + i * 16, 16)])
        plsc.subcore_barrier()

        @pl.when(active)
        def _scatter():
            for r in range(PA // 128):
                pltpu.sync_copy(feat.at[pl.ds(base + r * 128, nb[r]),
                                        pl.ds(col0, W)],
                                featbuf.at[pl.ds(0, nb[r])])
                pltpu.sync_copy(featbuf, grid_sh.at[idx_sc.at[r]],
                                add=True)
        plsc.subcore_barrier()

        @pl.when(active)
        def _gather():
            for r in range(PA // 128):
                pltpu.sync_copy(grid_sh.at[idx_ga.at[r]], featbuf)

                def _scale(j, _, r=r):
                    rv = precip[pl.ds(r * 128 + j, 16)][0]
                    for c in range(W // 16):
                        featbuf[j, pl.ds(c * 16, 16)] = (
                            featbuf[j, pl.ds(c * 16, 16)] * rv)
                    return 0
                lax.fori_loop(0, nb[r], _scale, 0)
                pltpu.sync_copy(featbuf.at[pl.ds(0, nb[r])],
                                out.at[pl.ds(base + r * 128, nb[r]),
                                       pl.ds(col0, W)])


@functools.partial(
    pl.kernel,
    out_type=jax.ShapeDtypeStruct((N, C), jnp.float32),
    mesh=plsc.VectorSubcoreMesh(core_axis_name="c", subcore_axis_name="s"),
    compiler_params=pltpu.CompilerParams(needs_layout_passes=False),
    scratch_types=[
        pltpu.VMEM_SHARED((GT, W), jnp.float32),
        pltpu.VMEM_SHARED((NT, NT, 256), jnp.int32),
        pltpu.VMEM_SHARED((G,), jnp.float32),
        pltpu.VMEM((128, W), jnp.float32),
        pltpu.VMEM((16, W), jnp.float32),
        pltpu.VMEM((P * 3,), jnp.float32),
        pltpu.VMEM((PA // 128, 128), jnp.int32),
        pltpu.VMEM((PA // 128, 128), jnp.int32),
        pltpu.VMEM((16, 256), jnp.int32),
        pltpu.VMEM((G,), jnp.float32),
        pltpu.VMEM((PA,), jnp.float32),
    ],
)
def _voxel_mean_sc(feat, coord, out, *scratch):
    _sc_body(feat, coord, out, *scratch)


def kernel(video_tensor, coord_info):
    feats = video_tensor.reshape(N, C)
    coords = coord_info.reshape(N * 3)
    out = _voxel_mean_sc(feats, coords)
    return out.reshape(V, L, C)


# speedup vs baseline: 1.1742x; 1.0507x over previous
import functools

import jax
import jax.numpy as jnp
from jax import lax
from jax.experimental import pallas as pl
from jax.experimental.pallas import tpu as pltpu
from jax.experimental.pallas import tpu_sc as plsc

V, L, C = 8, 729, 1152
LP = 736
N = V * LP
G = 4096
GT = G + 8
NT = 16
P = N // NT
PA = 384
NV = P // 16
W = 128
NCHUNK = C // W


def _sc_body(feat, coord, out, grid_sh, hist_sh, recip_sh,
             featbuf, zerobuf, coordbuf, idx_sc, idx_ga, hist_v, recip_v,
             precip):
    core = lax.axis_index("c")
    sid = lax.axis_index("s")
    lanes = lax.iota(jnp.int32, 16)

    base = sid * P

    zero16i = jnp.zeros((16,), jnp.int32)

    def _zero_hist(i, _):
        hist_v[i & 15, pl.ds((i >> 4) * 16, 16)] = zero16i
        return 0
    lax.fori_loop(0, 256, _zero_hist, 0)

    zero16f = jnp.zeros((16,), jnp.float32)
    for i in range(16):
        for c in range(W // 16):
            zerobuf[i, pl.ds(c * 16, 16)] = zero16f

    pltpu.sync_copy(coord.at[pl.ds(base, P)], coordbuf)

    ones_i = jnp.ones((16,), jnp.int32)
    for j in range(NV):
        pid = j * 16 + lanes
        fx = plsc.load_gather(coordbuf, [pid, jnp.zeros((16,), jnp.int32)])
        fy = plsc.load_gather(coordbuf, [pid, jnp.ones((16,), jnp.int32)])
        fz = plsc.load_gather(coordbuf, [pid, jnp.full((16,), 2, jnp.int32)])
        iv = (fx.astype(jnp.int32) * 256 + fy.astype(jnp.int32) * 16
              + fz.astype(jnp.int32))
        valid = jnp.remainder(base + pid, LP) < L
        iv_sc = jnp.where(valid, iv, G + (lanes & 7))
        row, col = j // 8, (j % 8) * 16
        idx_sc[row, pl.ds(col, 16)] = iv_sc
        idx_ga[row, pl.ds(col, 16)] = iv_sc
        plsc.addupdate_scatter(hist_v, [iv >> 8, iv & 255], ones_i,
                               mask=valid)
    idx_sc[2, pl.ds(112, 16)] = G + (lanes & 7)
    idx_ga[2, pl.ds(112, 16)] = G + (lanes & 7)

    pltpu.sync_copy(hist_v, hist_sh.at[sid])
    plsc.subcore_barrier()
    pltpu.sync_copy(hist_sh.at[:, sid], hist_v)
    for v in range(16):
        acc = hist_v[0, pl.ds(v * 16, 16)]
        for r in range(1, 16):
            acc = acc + hist_v[r, pl.ds(v * 16, 16)]
        cnt = jnp.maximum(acc.astype(jnp.float32), 1.0)
        precip[pl.ds(v * 16, 16)] = 1.0 / cnt
    pltpu.sync_copy(precip.at[pl.ds(0, 256)],
                    recip_sh.at[pl.ds(sid * 256, 256)])
    plsc.subcore_barrier()
    pltpu.sync_copy(recip_sh, recip_v)

    for j in range(NV):
        row, col = j // 8, (j % 8) * 16
        ivg = jnp.minimum(idx_ga[row, pl.ds(col, 16)], G - 1)
        precip[pl.ds(j * 16, 16)] = plsc.load_gather(recip_v, [ivg])

    nb = [128, 128, P - 256]
    for k in range((NCHUNK + 1) // 2):
        ch = k + core * 5
        active = ch < NCHUNK
        col0 = jnp.minimum(ch, NCHUNK - 1) * W
        plsc.subcore_barrier()

        @pl.when(active)
        def _zero():
            for i in range(16):
                pltpu.sync_copy(zerobuf,
                                grid_sh.at[pl.ds(sid * 256 + i * 16, 16)])
        plsc.subcore_barrier()

        @pl.when(active)
        def _scatter():
            for r in range(PA // 128):
                pltpu.sync_copy(feat.at[pl.ds(base + r * 128, nb[r]),
                                        pl.ds(col0, W)],
                                featbuf.at[pl.ds(0, nb[r])])
                pltpu.sync_copy(featbuf, grid_sh.at[idx_sc.at[r]],
                                add=True)
        plsc.subcore_barrier()

        @pl.when(active)
        def _gather():
            for r in range(PA // 128):
                pltpu.sync_copy(grid_sh.at[idx_ga.at[r]], featbuf)

                def _scale(j, _, r=r):
                    rv = precip[pl.ds(r * 128 + j, 16)][0]
                    for c in range(W // 16):
                        featbuf[j, pl.ds(c * 16, 16)] = (
                            featbuf[j, pl.ds(c * 16, 16)] * rv)
                    return 0
                lax.fori_loop(0, nb[r], _scale, 0)
                pltpu.sync_copy(featbuf.at[pl.ds(0, nb[r])],
                                out.at[pl.ds(base + r * 128, nb[r]),
                                       pl.ds(col0, W)])


@functools.partial(
    pl.kernel,
    out_type=jax.ShapeDtypeStruct((N, C), jnp.float32),
    mesh=plsc.VectorSubcoreMesh(core_axis_name="c", subcore_axis_name="s"),
    compiler_params=pltpu.CompilerParams(needs_layout_passes=False),
    scratch_types=[
        pltpu.VMEM_SHARED((GT, W), jnp.float32),
        pltpu.VMEM_SHARED((NT, NT, 256), jnp.int32),
        pltpu.VMEM_SHARED((G,), jnp.float32),
        pltpu.VMEM((128, W), jnp.float32),
        pltpu.VMEM((16, W), jnp.float32),
        pltpu.VMEM((P, 3), jnp.float32),
        pltpu.VMEM((PA // 128, 128), jnp.int32),
        pltpu.VMEM((PA // 128, 128), jnp.int32),
        pltpu.VMEM((16, 256), jnp.int32),
        pltpu.VMEM((G,), jnp.float32),
        pltpu.VMEM((PA,), jnp.float32),
    ],
)
def _voxel_mean_sc(feat, coord, out, *scratch):
    _sc_body(feat, coord, out, *scratch)


def kernel(video_tensor, coord_info):
    feats = jnp.pad(video_tensor, ((0, 0), (0, LP - L), (0, 0)))
    feats = feats.reshape(N, C)
    coords = jnp.pad(coord_info.reshape(V, L, 3), ((0, 0), (0, LP - L),
                                                   (0, 0)))
    coords = coords.reshape(N, 3)
    out = _voxel_mean_sc(feats, coords)
    return out.reshape(V, LP, C)[:, :L, :]


# speedup vs baseline: 2.0402x; 1.7375x over previous
import functools

import jax
import jax.numpy as jnp
from jax import lax
from jax.experimental import pallas as pl
from jax.experimental.pallas import tpu as pltpu
from jax.experimental.pallas import tpu_sc as plsc

V, L, C = 8, 729, 1152
LP = 736
N = V * LP
G = 4096
GT = G + 8
NT = 16
P = N // NT
PA = 384
NV = P // 16
W = 128
NCHUNK = C // W
KPC = 5
NB = [128, 128, P - 256]


def _sc_body(feat, coord, out, grid_sh, hist_sh, recip_sh,
             fb0, fb1, fb2, zerobuf, coordbuf, idx_sc, idx_ga, hist_v,
             recip_v, precip, zsem, lsem, ssem, gsem, tsem):
    core = lax.axis_index("c")
    sid = lax.axis_index("s")
    lanes = lax.iota(jnp.int32, 16)
    fb = [fb0, fb1, fb2]

    base = sid * P
    gbase = sid * 256

    zero16f = jnp.zeros((16,), jnp.float32)
    for i in range(32):
        for c in range(W // 16):
            zerobuf[i, pl.ds(c * 16, 16)] = zero16f

    def fire_zeros():
        return [pltpu.async_copy(zerobuf,
                                 grid_sh.at[pl.ds(gbase + i * 32, 32)],
                                 zsem)
                for i in range(8)]

    def fire_load(k, r):
        col0 = (core * (KPC - 1) + k) * W
        return pltpu.async_copy(
            feat.at[pl.ds(base + r * 128, NB[r]), pl.ds(col0, W)],
            fb[r].at[pl.ds(0, NB[r])], lsem.at[r])

    zd = fire_zeros()
    ld = [fire_load(0, r) for r in range(3)]

    zero16i = jnp.zeros((16,), jnp.int32)

    def _zero_hist(i, _):
        hist_v[i & 15, pl.ds((i >> 4) * 16, 16)] = zero16i
        return 0
    lax.fori_loop(0, 256, _zero_hist, 0)

    pltpu.sync_copy(coord.at[pl.ds(base * 3, P * 3)], coordbuf)

    ones_i = jnp.ones((16,), jnp.int32)
    for j in range(NV):
        pid = j * 16 + lanes
        cb = pid * 3
        fx = plsc.load_gather(coordbuf, [cb])
        fy = plsc.load_gather(coordbuf, [cb + 1])
        fz = plsc.load_gather(coordbuf, [cb + 2])
        iv = (fx.astype(jnp.int32) * 256 + fy.astype(jnp.int32) * 16
              + fz.astype(jnp.int32))
        valid = jnp.remainder(base + pid, LP) < L
        iv_sc = jnp.where(valid, iv, G + (lanes & 7))
        row, col = j // 8, (j % 8) * 16
        idx_sc[row, pl.ds(col, 16)] = iv_sc
        idx_ga[row, pl.ds(col, 16)] = iv_sc
        plsc.addupdate_scatter(hist_v, [iv >> 8, iv & 255], ones_i,
                               mask=valid)
    idx_sc[2, pl.ds(112, 16)] = G + (lanes & 7)
    idx_ga[2, pl.ds(112, 16)] = G + (lanes & 7)

    pltpu.sync_copy(hist_v, hist_sh.at[sid])
    plsc.subcore_barrier()
    pltpu.sync_copy(hist_sh.at[:, sid], hist_v)
    for q in range(16):
        acc = hist_v[0, pl.ds(q * 16, 16)]
        for r in range(1, 16):
            acc = acc + hist_v[r, pl.ds(q * 16, 16)]
        cnt = jnp.maximum(acc.astype(jnp.float32), 1.0)
        precip[pl.ds(q * 16, 16)] = 1.0 / cnt
    pltpu.sync_copy(precip.at[pl.ds(0, 256)],
                    recip_sh.at[pl.ds(sid * 256, 256)])
    plsc.subcore_barrier()
    pltpu.sync_copy(recip_sh, recip_v)

    for j in range(NV):
        row, col = j // 8, (j % 8) * 16
        ivg = jnp.minimum(idx_ga[row, pl.ds(col, 16)], G - 1)
        precip[pl.ds(j * 16, 16)] = plsc.load_gather(recip_v, [ivg])

    def scale_batch(r, nrows):
        def blk(b, _):
            rv = precip[pl.ds(r * 128 + b * 16, 16)]
            for i in range(16):
                s = rv[i]
                for c in range(W // 16):
                    fb[r][b * 16 + i, pl.ds(c * 16, 16)] = (
                        fb[r][b * 16 + i, pl.ds(c * 16, 16)] * s)
            return 0
        lax.fori_loop(0, nrows // 16, blk, 0)

    st = None
    for k in range(KPC):
        col0 = (core * (KPC - 1) + k) * W
        if k > 0:
            for r in range(3):
                st[r].wait()
            ld = [fire_load(k, r) for r in range(3)]
        for d in zd:
            d.wait()
        plsc.subcore_barrier()
        sc = []
        for r in range(3):
            ld[r].wait()
            sc.append(pltpu.async_copy(fb[r], grid_sh.at[idx_sc.at[r]],
                                       ssem, add=True))
        for d in sc:
            d.wait()
        plsc.subcore_barrier()
        g = [pltpu.async_copy(grid_sh.at[idx_ga.at[r]], fb[r], gsem.at[r])
             for r in range(3)]
        g[0].wait()
        scale_batch(0, NB[0])
        st0 = pltpu.async_copy(fb[0].at[pl.ds(0, NB[0])],
                               out.at[pl.ds(base, NB[0]), pl.ds(col0, W)],
                               tsem)
        g[1].wait()
        scale_batch(1, NB[1])
        st1 = pltpu.async_copy(fb[1].at[pl.ds(0, NB[1])],
                               out.at[pl.ds(base + 128, NB[1]),
                                      pl.ds(col0, W)], tsem)
        g[2].wait()
        plsc.subcore_barrier()
        if k + 1 < KPC:
            zd = fire_zeros()
        scale_batch(2, NB[2])
        st2 = pltpu.async_copy(fb[2].at[pl.ds(0, NB[2])],
                               out.at[pl.ds(base + 256, NB[2]),
                                      pl.ds(col0, W)], tsem)
        st = [st0, st1, st2]
    for r in range(3):
        st[r].wait()


@functools.partial(
    pl.kernel,
    out_type=jax.ShapeDtypeStruct((N, C), jnp.float32),
    mesh=plsc.VectorSubcoreMesh(core_axis_name="c", subcore_axis_name="s"),
    compiler_params=pltpu.CompilerParams(needs_layout_passes=False),
    scratch_types=[
        pltpu.VMEM_SHARED((GT, W), jnp.float32),
        pltpu.VMEM_SHARED((NT, NT, 256), jnp.int32),
        pltpu.VMEM_SHARED((G,), jnp.float32),
        pltpu.VMEM((128, W), jnp.float32),
        pltpu.VMEM((128, W), jnp.float32),
        pltpu.VMEM((128, W), jnp.float32),
        pltpu.VMEM((32, W), jnp.float32),
        pltpu.VMEM((P * 3,), jnp.float32),
        pltpu.VMEM((PA // 128, 128), jnp.int32),
        pltpu.VMEM((PA // 128, 128), jnp.int32),
        pltpu.VMEM((16, 256), jnp.int32),
        pltpu.VMEM((G,), jnp.float32),
        pltpu.VMEM((PA,), jnp.float32),
        pltpu.SemaphoreType.DMA,
        pltpu.SemaphoreType.DMA((3,)),
        pltpu.SemaphoreType.DMA,
        pltpu.SemaphoreType.DMA((3,)),
        pltpu.SemaphoreType.DMA,
    ],
)
def _voxel_mean_sc(feat, coord, out, *scratch):
    _sc_body(feat, coord, out, *scratch)


def kernel(video_tensor, coord_info):
    vt = video_tensor
    feats = jnp.pad(vt, ((0, 0), (0, LP - L), (0, 0)))
    feats = feats.reshape(N, C)
    coords = jnp.pad(coord_info.reshape(V, L, 3), ((0, 0), (0, LP - L),
                                                   (0, 0)))
    coords = coords.reshape(N * 3)
    out = _voxel_mean_sc(feats, coords)
    out = out.reshape(V, LP, C)[:, :L, :]
    zout = lax.optimization_barrier(jnp.zeros((), jnp.float32))
    return out + zout
